# baseline (speedup 1.0000x reference)
import jax
import jax.numpy as jnp
from jax.experimental import pallas as pl
from jax.experimental.pallas import tpu as pltpu

_B, _C, _N = 64, 1024, 256
_NHID, _NCLASS, _K = 1024, 1024, 10


def _dot(a, b, dims, prec):
    return jax.lax.dot_general(a, b, dims,
                               preferred_element_type=jnp.float32,
                               precision=prec)


def _gcn_body(v_ref, g_w_ref, g_b_ref, w1t_ref, b1_ref, w2t_ref, b2_ref,
              out_ref):
    f32 = jnp.float32
    prec = jax.lax.Precision.HIGHEST
    v_b = v_ref[0]

    A = _dot(v_b, v_b, (((0,), (0,)), ((), ())), prec)

    neg = jnp.float32(-jnp.inf)
    cur = A
    total = jnp.zeros((1, _N), f32)
    thresh = jnp.zeros((1, _N), f32)
    keep_at = jnp.zeros((1, _N), f32)
    for _ in range(_K):
        t = jnp.max(cur, axis=0, keepdims=True)
        c = jnp.sum((cur == t).astype(f32), axis=0, keepdims=True)
        done_now = jnp.logical_and(total < _K, total + c >= _K)
        thresh = jnp.where(done_now, t, thresh)
        keep_at = jnp.where(done_now, _K - total, keep_at)
        total = total + c
        cur = jnp.where(cur == t, neg, cur)

    eq = (A == thresh)
    gt = A > thresh
    ri = jax.lax.broadcasted_iota(jnp.int32, (_N, _N), 0)
    ci = jax.lax.broadcasted_iota(jnp.int32, (_N, _N), 1)
    L = (ci <= ri).astype(f32)
    cum = _dot(L, eq.astype(f32), (((1,), (0,)), ((), ())), prec)
    keep = jnp.logical_or(gt, jnp.logical_and(eq, cum <= keep_at))
    eye = (ri == ci).astype(f32)
    amt = jnp.where(keep, A, 0.0) + eye

    deg_row = jnp.sum(amt, axis=0, keepdims=True)
    ones_col = jnp.ones((_N, 1), f32)
    deg_col = _dot(amt, ones_col, (((0,), (0,)), ((), ())), prec)
    dr = jax.lax.rsqrt(deg_row)
    dr = jnp.where(jnp.isinf(dr), 0.0, dr)
    dc = jax.lax.rsqrt(deg_col)
    dc = jnp.where(jnp.isinf(dc), 0.0, dc)
    a_hat_t = dc * amt * dr

    gv = _dot(g_w_ref[...], v_b, (((1,), (0,)), ((), ())), prec) + g_b_ref[...]
    h1 = _dot(w1t_ref[...], gv, (((1,), (0,)), ((), ())), prec)
    x = jnp.maximum(_dot(h1, a_hat_t, (((1,), (0,)), ((), ())), prec) + b1_ref[...],
                    0.0)
    h2 = _dot(w2t_ref[...], x, (((1,), (0,)), ((), ())), prec)
    y = _dot(h2, a_hat_t, (((1,), (0,)), ((), ())), prec) + b2_ref[...]
    out_ref[0] = y + v_b


def kernel(v, g_w, g_b, gc1_w, gc1_b, gc2_w, gc2_b):
    f32 = jnp.float32
    w1t = gc1_w.T
    w2t = gc2_w.T
    g_b2 = g_b.reshape(_NHID, 1)
    b1 = gc1_b.reshape(_NHID, 1)
    b2 = gc2_b.reshape(_NCLASS, 1)

    full = lambda shape: pl.BlockSpec(shape, lambda b: (0,) * len(shape))
    return pl.pallas_call(
        _gcn_body,
        grid=(_B,),
        in_specs=[
            pl.BlockSpec((1, _C, _N), lambda b: (b, 0, 0)),
            full((_NHID, _C)),
            full((_NHID, 1)),
            full((_NHID, _C)),
            full((_NHID, 1)),
            full((_NCLASS, _NHID)),
            full((_NCLASS, 1)),
        ],
        out_specs=pl.BlockSpec((1, _NCLASS, _N), lambda b: (b, 0, 0)),
        out_shape=jax.ShapeDtypeStruct((_B, _NCLASS, _N), f32),
        compiler_params=pltpu.CompilerParams(
            dimension_semantics=("arbitrary",)),
    )(v, g_w, g_b2, w1t, b1, w2t, b2)


# speedup vs baseline: 3.2984x; 3.2984x over previous
import jax
import jax.numpy as jnp
from jax.experimental import pallas as pl
from jax.experimental.pallas import tpu as pltpu

_B, _C, _N = 64, 1024, 256
_NHID, _NCLASS, _K = 1024, 1024, 10


def _dot(a, b, dims, prec):
    return jax.lax.dot_general(a, b, dims,
                               preferred_element_type=jnp.float32,
                               precision=prec)


def _gcn_body(v_ref, g_w_ref, g_b_ref, w1t_ref, b1_ref, w2t_ref, b2_ref,
              out_ref):
    f32 = jnp.float32
    prec = jax.lax.Precision.HIGHEST
    fast = jax.lax.Precision.DEFAULT
    v_b = v_ref[0]

    A = _dot(v_b, v_b, (((0,), (0,)), ((), ())), prec)

    neg = jnp.float32(-jnp.inf)
    cur = A
    total = jnp.zeros((1, _N), f32)
    thresh = jnp.zeros((1, _N), f32)
    keep_at = jnp.zeros((1, _N), f32)
    for _ in range(_K):
        t = jnp.max(cur, axis=0, keepdims=True)
        c = jnp.sum((cur == t).astype(f32), axis=0, keepdims=True)
        done_now = jnp.logical_and(total < _K, total + c >= _K)
        thresh = jnp.where(done_now, t, thresh)
        keep_at = jnp.where(done_now, _K - total, keep_at)
        total = total + c
        cur = jnp.where(cur == t, neg, cur)

    eq = (A == thresh)
    gt = A > thresh
    ri = jax.lax.broadcasted_iota(jnp.int32, (_N, _N), 0)
    ci = jax.lax.broadcasted_iota(jnp.int32, (_N, _N), 1)
    L = (ci <= ri).astype(f32)
    cum = _dot(L, eq.astype(f32), (((1,), (0,)), ((), ())), prec)
    keep = jnp.logical_or(gt, jnp.logical_and(eq, cum <= keep_at))
    eye = (ri == ci).astype(f32)
    amt = jnp.where(keep, A, 0.0) + eye

    deg_row = jnp.sum(amt, axis=0, keepdims=True)
    ones_col = jnp.ones((_N, 1), f32)
    deg_col = _dot(amt, ones_col, (((0,), (0,)), ((), ())), prec)
    dr = jax.lax.rsqrt(deg_row)
    dr = jnp.where(jnp.isinf(dr), 0.0, dr)
    dc = jax.lax.rsqrt(deg_col)
    dc = jnp.where(jnp.isinf(dc), 0.0, dc)
    a_hat_t = dc * amt * dr

    gv = _dot(g_w_ref[...], v_b, (((1,), (0,)), ((), ())), fast) + g_b_ref[...]
    h1 = _dot(w1t_ref[...], gv, (((1,), (0,)), ((), ())), fast)
    x = jnp.maximum(_dot(h1, a_hat_t, (((1,), (0,)), ((), ())), fast) + b1_ref[...],
                    0.0)
    h2 = _dot(w2t_ref[...], x, (((1,), (0,)), ((), ())), fast)
    y = _dot(h2, a_hat_t, (((1,), (0,)), ((), ())), fast) + b2_ref[...]
    out_ref[0] = y + v_b


def kernel(v, g_w, g_b, gc1_w, gc1_b, gc2_w, gc2_b):
    f32 = jnp.float32
    w1t = gc1_w.T
    w2t = gc2_w.T
    g_b2 = g_b.reshape(_NHID, 1)
    b1 = gc1_b.reshape(_NHID, 1)
    b2 = gc2_b.reshape(_NCLASS, 1)

    full = lambda shape: pl.BlockSpec(shape, lambda b: (0,) * len(shape))
    return pl.pallas_call(
        _gcn_body,
        grid=(_B,),
        in_specs=[
            pl.BlockSpec((1, _C, _N), lambda b: (b, 0, 0)),
            full((_NHID, _C)),
            full((_NHID, 1)),
            full((_NHID, _C)),
            full((_NHID, 1)),
            full((_NCLASS, _NHID)),
            full((_NCLASS, 1)),
        ],
        out_specs=pl.BlockSpec((1, _NCLASS, _N), lambda b: (b, 0, 0)),
        out_shape=jax.ShapeDtypeStruct((_B, _NCLASS, _N), f32),
        compiler_params=pltpu.CompilerParams(
            dimension_semantics=("arbitrary",)),
    )(v, g_w, g_b2, w1t, b1, w2t, b2)


# speedup vs baseline: 5.4301x; 1.6463x over previous
import jax
import jax.numpy as jnp
from jax.experimental import pallas as pl
from jax.experimental.pallas import tpu as pltpu

_B, _C, _N = 64, 1024, 256
_NHID, _NCLASS, _K = 1024, 1024, 10


def _dot(a, b, dims, prec):
    return jax.lax.dot_general(a, b, dims,
                               preferred_element_type=jnp.float32,
                               precision=prec)


def _fold_body(w1t_ref, g_w_ref, g_b_ref, m_ref, c1_ref):
    prec = jax.lax.Precision.DEFAULT
    m_ref[...] = _dot(w1t_ref[...], g_w_ref[...],
                      (((1,), (0,)), ((), ())), prec)
    c1_ref[...] = _dot(w1t_ref[...], g_b_ref[...],
                       (((1,), (0,)), ((), ())), prec)


def _gcn_body(v_ref, m_ref, c1_ref, b1_ref, w2t_ref, b2_ref, out_ref):
    f32 = jnp.float32
    prec = jax.lax.Precision.DEFAULT
    v_b = v_ref[0]

    A = _dot(v_b, v_b, (((0,), (0,)), ((), ())), prec)

    neg = jnp.float32(-jnp.inf)
    cur = A
    total = jnp.zeros((1, _N), f32)
    thresh = jnp.zeros((1, _N), f32)
    keep_at = jnp.zeros((1, _N), f32)
    for _ in range(_K):
        t = jnp.max(cur, axis=0, keepdims=True)
        c = jnp.sum((cur == t).astype(f32), axis=0, keepdims=True)
        done_now = jnp.logical_and(total < _K, total + c >= _K)
        thresh = jnp.where(done_now, t, thresh)
        keep_at = jnp.where(done_now, _K - total, keep_at)
        total = total + c
        cur = jnp.where(cur == t, neg, cur)

    eq = (A == thresh)
    gt = A > thresh
    ri = jax.lax.broadcasted_iota(jnp.int32, (_N, _N), 0)
    ci = jax.lax.broadcasted_iota(jnp.int32, (_N, _N), 1)
    L = (ci <= ri).astype(f32)
    cum = _dot(L, eq.astype(f32), (((1,), (0,)), ((), ())), prec)
    keep = jnp.logical_or(gt, jnp.logical_and(eq, cum <= keep_at))
    eye = (ri == ci).astype(f32)
    amt = jnp.where(keep, A, 0.0) + eye

    deg_row = jnp.sum(amt, axis=0, keepdims=True)
    ones_col = jnp.ones((_N, 1), f32)
    deg_col = _dot(amt, ones_col, (((0,), (0,)), ((), ())), prec)
    dr = jax.lax.rsqrt(deg_row)
    dr = jnp.where(jnp.isinf(dr), 0.0, dr)
    dc = jax.lax.rsqrt(deg_col)
    dc = jnp.where(jnp.isinf(dc), 0.0, dc)
    a_hat_t = dc * amt * dr

    h1 = _dot(m_ref[...], v_b, (((1,), (0,)), ((), ())), prec) + c1_ref[...]
    x = jnp.maximum(
        _dot(h1, a_hat_t, (((1,), (0,)), ((), ())), prec) + b1_ref[...], 0.0)
    h2 = _dot(w2t_ref[...], x, (((1,), (0,)), ((), ())), prec)
    y = _dot(h2, a_hat_t, (((1,), (0,)), ((), ())), prec) + b2_ref[...]
    out_ref[0] = y + v_b


def kernel(v, g_w, g_b, gc1_w, gc1_b, gc2_w, gc2_b):
    f32 = jnp.float32
    w1t = gc1_w.T
    w2t = gc2_w.T
    g_b2 = g_b.reshape(_C, 1)
    b1 = gc1_b.reshape(_NHID, 1)
    b2 = gc2_b.reshape(_NCLASS, 1)

    m, c1 = pl.pallas_call(
        _fold_body,
        out_shape=(jax.ShapeDtypeStruct((_NHID, _C), f32),
                   jax.ShapeDtypeStruct((_NHID, 1), f32)),
    )(w1t, g_w, g_b2)

    full = lambda shape: pl.BlockSpec(shape, lambda b: (0,) * len(shape))
    return pl.pallas_call(
        _gcn_body,
        grid=(_B,),
        in_specs=[
            pl.BlockSpec((1, _C, _N), lambda b: (b, 0, 0)),
            full((_NHID, _C)),
            full((_NHID, 1)),
            full((_NHID, 1)),
            full((_NCLASS, _NHID)),
            full((_NCLASS, 1)),
        ],
        out_specs=pl.BlockSpec((1, _NCLASS, _N), lambda b: (b, 0, 0)),
        out_shape=jax.ShapeDtypeStruct((_B, _NCLASS, _N), f32),
        compiler_params=pltpu.CompilerParams(
            dimension_semantics=("arbitrary",)),
    )(v, m, c1, b1, w2t, b2)


# speedup vs baseline: 5.4722x; 1.0078x over previous
import jax
import jax.numpy as jnp
from jax.experimental import pallas as pl
from jax.experimental.pallas import tpu as pltpu

_B, _C, _N = 64, 1024, 256
_NHID, _NCLASS, _K = 1024, 1024, 10


def _dot(a, b, dims, prec):
    return jax.lax.dot_general(a, b, dims,
                               preferred_element_type=jnp.float32,
                               precision=prec)


def _fold_body(w1t_ref, g_w_ref, g_b_ref, m_ref, c1_ref):
    prec = jax.lax.Precision.DEFAULT
    m_ref[...] = _dot(w1t_ref[...], g_w_ref[...],
                      (((1,), (0,)), ((), ())), prec).astype(jnp.bfloat16)
    c1_ref[...] = _dot(w1t_ref[...], g_b_ref[...],
                       (((1,), (0,)), ((), ())), prec)


def _gcn_body(v_ref, m_ref, c1_ref, b1_ref, w2t_ref, b2_ref, out_ref):
    f32 = jnp.float32
    prec = jax.lax.Precision.DEFAULT
    v_b = v_ref[0]

    A = _dot(v_b, v_b, (((0,), (0,)), ((), ())), prec)

    neg = jnp.float32(-jnp.inf)
    cur = A
    total = jnp.zeros((1, _N), f32)
    thresh = jnp.zeros((1, _N), f32)
    keep_at = jnp.zeros((1, _N), f32)
    for _ in range(_K):
        t = jnp.max(cur, axis=0, keepdims=True)
        c = jnp.sum((cur == t).astype(f32), axis=0, keepdims=True)
        done_now = jnp.logical_and(total < _K, total + c >= _K)
        thresh = jnp.where(done_now, t, thresh)
        keep_at = jnp.where(done_now, _K - total, keep_at)
        total = total + c
        cur = jnp.where(cur == t, neg, cur)

    eq = (A == thresh)
    gt = A > thresh
    ri = jax.lax.broadcasted_iota(jnp.int32, (_N, _N), 0)
    ci = jax.lax.broadcasted_iota(jnp.int32, (_N, _N), 1)
    L = (ci <= ri).astype(f32)
    cum = _dot(L, eq.astype(f32), (((1,), (0,)), ((), ())), prec)
    keep = jnp.logical_or(gt, jnp.logical_and(eq, cum <= keep_at))
    eye = (ri == ci).astype(f32)
    amt = jnp.where(keep, A, 0.0) + eye

    deg_row = jnp.sum(amt, axis=0, keepdims=True)
    ones_col = jnp.ones((_N, 1), f32)
    deg_col = _dot(amt, ones_col, (((0,), (0,)), ((), ())), prec)
    dr = jax.lax.rsqrt(deg_row)
    dr = jnp.where(jnp.isinf(dr), 0.0, dr)
    dc = jax.lax.rsqrt(deg_col)
    dc = jnp.where(jnp.isinf(dc), 0.0, dc)
    a_hat_t = dc * amt * dr

    bf16 = jnp.bfloat16
    ah16 = a_hat_t.astype(bf16)
    h1 = _dot(m_ref[...], v_b.astype(bf16),
              (((1,), (0,)), ((), ())), prec) + c1_ref[...]
    x = jnp.maximum(
        _dot(h1.astype(bf16), ah16, (((1,), (0,)), ((), ())), prec)
        + b1_ref[...], 0.0)
    h2 = _dot(w2t_ref[...], x.astype(bf16), (((1,), (0,)), ((), ())), prec)
    y = _dot(h2.astype(bf16), ah16, (((1,), (0,)), ((), ())), prec) \
        + b2_ref[...]
    out_ref[0] = y + v_b


def kernel(v, g_w, g_b, gc1_w, gc1_b, gc2_w, gc2_b):
    f32 = jnp.float32
    w1t = gc1_w.T
    w2t = gc2_w.T.astype(jnp.bfloat16)
    g_b2 = g_b.reshape(_C, 1)
    b1 = gc1_b.reshape(_NHID, 1)
    b2 = gc2_b.reshape(_NCLASS, 1)

    m, c1 = pl.pallas_call(
        _fold_body,
        out_shape=(jax.ShapeDtypeStruct((_NHID, _C), jnp.bfloat16),
                   jax.ShapeDtypeStruct((_NHID, 1), f32)),
    )(w1t, g_w, g_b2)

    full = lambda shape: pl.BlockSpec(shape, lambda b: (0,) * len(shape))
    return pl.pallas_call(
        _gcn_body,
        grid=(_B,),
        in_specs=[
            pl.BlockSpec((1, _C, _N), lambda b: (b, 0, 0)),
            full((_NHID, _C)),
            full((_NHID, 1)),
            full((_NHID, 1)),
            full((_NCLASS, _NHID)),
            full((_NCLASS, 1)),
        ],
        out_specs=pl.BlockSpec((1, _NCLASS, _N), lambda b: (b, 0, 0)),
        out_shape=jax.ShapeDtypeStruct((_B, _NCLASS, _N), f32),
        compiler_params=pltpu.CompilerParams(
            dimension_semantics=("arbitrary",)),
    )(v, m, c1, b1, w2t, b2)
